# baseline (speedup 1.0000x reference)
import functools

import jax
import jax.numpy as jnp
from jax import lax
from jax.experimental import pallas as pl
from jax.experimental.pallas import tpu as pltpu
from jax.experimental.pallas import tpu_sc as plsc

BATCH = 4096
SEQ = 200
DIM = 64
NW = 32
B_PER_W = BATCH // NW
C0 = 104
C1 = SEQ - C0


@functools.partial(
    pl.kernel,
    out_type=jax.ShapeDtypeStruct((BATCH * DIM,), jnp.float32),
    mesh=plsc.VectorSubcoreMesh(core_axis_name="c", subcore_axis_name="s"),
    scratch_types=[
        pltpu.VMEM((B_PER_W * SEQ,), jnp.int32),
        pltpu.VMEM((C0, DIM), jnp.float32),
        pltpu.VMEM((C1, DIM), jnp.float32),
        pltpu.VMEM((B_PER_W * DIM,), jnp.float32),
        pltpu.SemaphoreType.DMA,
        pltpu.SemaphoreType.DMA,
    ],
    compiler_params=pltpu.CompilerParams(needs_layout_passes=False,
                                         use_tc_tiling_on_sc=False),
)
def _fasttext_sc(table_hbm, idx_hbm, out_hbm, idx_v,
                 rows0, rows1, out_v, sem0, sem1):
    nc = 2
    wid = lax.axis_index("s") * nc + lax.axis_index("c")
    base = wid * B_PER_W

    pltpu.sync_copy(idx_hbm.at[pl.ds(base * SEQ, B_PER_W * SEQ)], idx_v)

    lane = lax.iota(jnp.int32, 16)
    cols = [lane + (16 * c) for c in range(4)]

    def accum(rows_ref, n, acc):
        def body(j, a):
            r = jnp.full((16,), j, jnp.int32)
            return tuple(a[c] + plsc.load_gather(rows_ref, [r, cols[c]])
                         for c in range(4))
        return lax.fori_loop(0, n, body, acc)

    def row_body(b, carry):
        off = pl.multiple_of(b * SEQ, 8)
        g0 = pltpu.async_copy(table_hbm.at[idx_v.at[pl.ds(off, C0)]],
                              rows0, sem0)
        g1 = pltpu.async_copy(table_hbm.at[idx_v.at[pl.ds(off + C0, C1)]],
                              rows1, sem1)
        g0.wait()
        g1.wait()
        z = jnp.zeros((16,), jnp.float32)
        acc = accum(rows0, C0, (z, z, z, z))
        acc = accum(rows1, C1, acc)
        scale = jnp.float32(1.0 / SEQ)
        obase = b * DIM
        for c in range(4):
            plsc.store_scatter(out_v, [obase + 16 * c + lane], acc[c] * scale)
        return carry

    lax.fori_loop(0, B_PER_W, row_body, 0)

    pltpu.sync_copy(out_v, out_hbm.at[pl.ds(base * DIM, B_PER_W * DIM)])


def kernel(input, table):
    idx_flat = input.astype(jnp.int32).reshape(BATCH * SEQ)
    out_flat = _fasttext_sc(table, idx_flat)
    return out_flat.reshape(BATCH, DIM)


# speedup vs baseline: 1.1810x; 1.1810x over previous
import functools

import jax
import jax.numpy as jnp
from jax import lax
from jax.experimental import pallas as pl
from jax.experimental.pallas import tpu as pltpu
from jax.experimental.pallas import tpu_sc as plsc

BATCH = 4096
SEQ = 200
DIM = 64
NW = 32
B_PER_W = BATCH // NW
C0 = 104
C1 = SEQ - C0


@functools.partial(
    pl.kernel,
    out_type=jax.ShapeDtypeStruct((BATCH * DIM,), jnp.float32),
    mesh=plsc.VectorSubcoreMesh(core_axis_name="c", subcore_axis_name="s"),
    scratch_types=[
        pltpu.VMEM((B_PER_W * SEQ,), jnp.int32),
        pltpu.VMEM((C0, DIM), jnp.float32),
        pltpu.VMEM((C1, DIM), jnp.float32),
        pltpu.VMEM((C0, DIM), jnp.float32),
        pltpu.VMEM((C1, DIM), jnp.float32),
        pltpu.VMEM((B_PER_W * DIM,), jnp.float32),
        pltpu.SemaphoreType.DMA,
        pltpu.SemaphoreType.DMA,
        pltpu.SemaphoreType.DMA,
        pltpu.SemaphoreType.DMA,
    ],
    compiler_params=pltpu.CompilerParams(needs_layout_passes=False,
                                         use_tc_tiling_on_sc=False),
)
def _fasttext_sc(table_hbm, idx_hbm, out_hbm, idx_v,
                 ra0, ra1, rb0, rb1, out_v, sa0, sa1, sb0, sb1):
    nc = 2
    wid = lax.axis_index("s") * nc + lax.axis_index("c")
    base = wid * B_PER_W

    pltpu.sync_copy(idx_hbm.at[pl.ds(base * SEQ, B_PER_W * SEQ)], idx_v)

    def start(b, r0, r1, s0, s1):
        off = pl.multiple_of(b * SEQ, 8)
        pltpu.async_copy(table_hbm.at[idx_v.at[pl.ds(off, C0)]], r0, s0)
        pltpu.async_copy(table_hbm.at[idx_v.at[pl.ds(off + C0, C1)]], r1, s1)

    def wait(r0, r1, s0, s1):
        pltpu.make_async_copy(table_hbm.at[idx_v.at[pl.ds(0, C0)]],
                              r0, s0).wait()
        pltpu.make_async_copy(table_hbm.at[idx_v.at[pl.ds(C0, C1)]],
                              r1, s1).wait()

    def accum(rows_ref, n, acc):
        def body(g, a):
            j0 = pl.multiple_of(g * 8, 8)
            for u in range(8):
                a = tuple(a[c] + rows_ref[j0 + u, pl.ds(16 * c, 16)]
                          for c in range(4))
            return a
        return lax.fori_loop(0, n // 8, body, acc)

    scale = jnp.float32(1.0 / SEQ)

    def accum_row(b, r0, r1):
        z = jnp.zeros((16,), jnp.float32)
        acc = accum(r0, C0, (z, z, z, z))
        acc = accum(r1, C1, acc)
        ob = pl.multiple_of(b * DIM, 8)
        for c in range(4):
            out_v[pl.ds(ob + 16 * c, 16)] = acc[c] * scale

    start(0, ra0, ra1, sa0, sa1)

    def pair_body(i, carry):
        b0 = 2 * i
        start(b0 + 1, rb0, rb1, sb0, sb1)
        wait(ra0, ra1, sa0, sa1)
        accum_row(b0, ra0, ra1)

        @pl.when(i < B_PER_W // 2 - 1)
        def _():
            start(b0 + 2, ra0, ra1, sa0, sa1)

        wait(rb0, rb1, sb0, sb1)
        accum_row(b0 + 1, rb0, rb1)
        return carry

    lax.fori_loop(0, B_PER_W // 2, pair_body, 0)

    pltpu.sync_copy(out_v, out_hbm.at[pl.ds(base * DIM, B_PER_W * DIM)])


def kernel(input, table):
    idx_flat = input.astype(jnp.int32).reshape(BATCH * SEQ)
    out_flat = _fasttext_sc(table, idx_flat)
    return out_flat.reshape(BATCH, DIM)
